# baseline (speedup 1.0000x reference)
import functools

import jax
import jax.numpy as jnp
from jax import lax
from jax.experimental import pallas as pl
from jax.experimental.pallas import tpu as pltpu
from jax.experimental.pallas import tpu_sc as plsc

GAMMA = 0.1
EPSILON = 0.1

NC = 2
NS = 16
K = 128
NG = 2
NI = 5


@functools.cache
def _sc_mesh():
    return plsc.VectorSubcoreMesh(core_axis_name="core",
                                  subcore_axis_name="subcore",
                                  num_cores=NC, num_subcores=NS)


def _deg_body(npad, nch_deg, ei_hbm, ones_hbm, zeros_hbm, degp_hbm,
              i0, i1, ones_v, zbuf, deg_sh, is0, is1):
    stripe = npad // NS
    c = lax.axis_index("core")
    s = lax.axis_index("subcore")
    pltpu.sync_copy(zeros_hbm, zbuf)
    pltpu.sync_copy(zbuf, deg_sh.at[pl.ds(s * stripe, stripe)])
    pltpu.sync_copy(ones_hbm, ones_v)
    plsc.subcore_barrier()

    base = (c * NS + s) * nch_deg * K
    pltpu.async_copy(ei_hbm.at[:, pl.ds(base, K)], i0, is0)
    pltpu.async_copy(ei_hbm.at[:, pl.ds(base + K, K)], i1, is1)

    @pl.loop(0, nch_deg, step=2)
    def _(j):
        pltpu.make_async_copy(ei_hbm.at[:, pl.ds(base, K)], i0, is0).wait()
        pltpu.sync_copy(ones_v, deg_sh.at[i0.at[1]], add=True)

        @pl.when(j + 2 < nch_deg)
        def _():
            pltpu.async_copy(ei_hbm.at[:, pl.ds(base + (j + 2) * K, K)],
                             i0, is0)

        pltpu.make_async_copy(ei_hbm.at[:, pl.ds(base, K)], i1, is1).wait()
        pltpu.sync_copy(ones_v, deg_sh.at[i1.at[1]], add=True)

        @pl.when(j + 3 < nch_deg)
        def _():
            pltpu.async_copy(ei_hbm.at[:, pl.ds(base + (j + 3) * K, K)],
                             i1, is1)

    plsc.subcore_barrier()
    pltpu.sync_copy(deg_sh.at[pl.ds(s * stripe, stripe)], zbuf)
    pltpu.sync_copy(zbuf, degp_hbm.at[pl.ds(c * npad + s * stripe, stripe)])


def _gcn_body(npad, nch, xws0_hbm, xws1_hbm, ei_hbm, zeros_hbm, gcn_hbm,
              ib, gb, acc_sh, isems, gsems, ssems):
    stripe = npad // NS
    c = lax.axis_index("core")
    s = lax.axis_index("subcore")
    pltpu.sync_copy(zeros_hbm, gb[0])

    @pl.loop(0, stripe, step=K)
    def _(i):
        pltpu.sync_copy(gb[0], acc_sh.at[pl.ds(s * stripe + i, K)])

    plsc.subcore_barrier()
    base = s * nch * K

    def edge_loop(xt):
        for t in range(3):
            pltpu.async_copy(ei_hbm.at[:, pl.ds(base + t * K, K)], ib[t],
                             isems[t])

        @pl.loop(0, nch, step=2 * NI)
        def _(j):
            for u in range(2 * NI):
                t = j + u
                b = u % NG
                i5 = u % NI
                p = (u - 1) % NG
                p5 = (u - 1) % NI
                if u >= 2:
                    pltpu.make_async_copy(gb[b], acc_sh.at[ib[i5].at[1]],
                                          ssems[b]).wait()
                else:
                    @pl.when(t >= 2)
                    def _():
                        pltpu.make_async_copy(gb[b],
                                              acc_sh.at[ib[i5].at[1]],
                                              ssems[b]).wait()
                pltpu.make_async_copy(ei_hbm.at[:, pl.ds(base, K)], ib[i5],
                                      isems[i5]).wait()
                pltpu.async_copy(xt.at[ib[i5].at[0]], gb[b], gsems[b])
                if u >= 1:
                    pltpu.make_async_copy(xt.at[ib[p5].at[0]], gb[p],
                                          gsems[p]).wait()
                    pltpu.async_copy(gb[p], acc_sh.at[ib[p5].at[1]],
                                     ssems[p], add=True)
                else:
                    @pl.when(t >= 1)
                    def _():
                        pltpu.make_async_copy(xt.at[ib[p5].at[0]], gb[p],
                                              gsems[p]).wait()
                        pltpu.async_copy(gb[p], acc_sh.at[ib[p5].at[1]],
                                         ssems[p], add=True)
                i3 = (u + 3) % NI

                @pl.when(t + 3 < nch)
                def _():
                    pltpu.async_copy(ei_hbm.at[:, pl.ds(base + (t + 3) * K, K)],
                                     ib[i3], isems[i3])

        pltpu.make_async_copy(xt.at[ib[(nch - 1) % NI].at[0]],
                              gb[(nch - 1) % NG],
                              gsems[(nch - 1) % NG]).wait()
        pltpu.sync_copy(gb[(nch - 1) % NG],
                        acc_sh.at[ib[(nch - 1) % NI].at[1]], add=True)
        pltpu.make_async_copy(gb[(nch - 2) % NG],
                              acc_sh.at[ib[(nch - 2) % NI].at[1]],
                              ssems[(nch - 2) % NG]).wait()

    @pl.when(c == 0)
    def _():
        edge_loop(xws0_hbm)

    @pl.when(c == 1)
    def _():
        edge_loop(xws1_hbm)

    plsc.subcore_barrier()

    nz = stripe // K
    obase = s * stripe
    pltpu.async_copy(acc_sh.at[pl.ds(obase, K)], gb[0], gsems[0])
    for k in range(nz):
        if k >= 1 and k + 1 < nz:
            pltpu.make_async_copy(gb[(k - 1) % NG],
                                  gcn_hbm.at[c, pl.ds(obase, K)],
                                  ssems[(k - 1) % NG]).wait()
        if k + 1 < nz:
            pltpu.async_copy(acc_sh.at[pl.ds(obase + (k + 1) * K, K)],
                             gb[(k + 1) % NG], gsems[(k + 1) % NG])
        pltpu.make_async_copy(acc_sh.at[pl.ds(obase, K)], gb[k % NG],
                              gsems[k % NG]).wait()
        pltpu.async_copy(gb[k % NG], gcn_hbm.at[c, pl.ds(obase + k * K, K)],
                         ssems[k % NG])
    for k in (nz - 2, nz - 1):
        pltpu.make_async_copy(gb[k % NG], gcn_hbm.at[c, pl.ds(obase, K)],
                              ssems[k % NG]).wait()


def _dense_body(x_ref, wcat_ref, degp0_ref, degp1_ref, h2_ref, xws0_ref,
                xws1_ref):
    xb = x_ref[...]
    m = jnp.dot(xb.astype(jnp.bfloat16), wcat_ref[...].astype(jnp.bfloat16),
                preferred_element_type=jnp.float32)
    d = xb.shape[1]
    h2_ref[...] = m[:, d:]
    deg = degp0_ref[...] + degp1_ref[...]
    dis = jnp.where(deg > 0.0, lax.rsqrt(deg), 0.0)
    xw = m[:, :d] * dis
    half = d // 2
    xws0_ref[...] = xw[:, :half]
    xws1_ref[...] = xw[:, half:]


def _combine_body(x_ref, h2_ref, gcn_ref, degp0_ref, degp1_ref, bias_ref,
                  o_ref):
    deg = degp0_ref[...] + degp1_ref[...]
    dis = jnp.where(deg > 0.0, lax.rsqrt(deg), 0.0)
    g = jnp.concatenate([gcn_ref[0], gcn_ref[1]], axis=1)
    h = h2_ref[...] + g * dis + bias_ref[...]
    o_ref[...] = x_ref[...] + EPSILON * jnp.tanh(h)


def kernel(x, edge_index, W, W_phi, bias):
    n, d = x.shape
    e = edge_index.shape[1]
    half = d // 2
    npad = ((n + K * NS - 1) // (K * NS)) * (K * NS)
    stripe = npad // NS
    nch = 2 * NI * ((e + 2 * NI * NS * K - 1) // (2 * NI * NS * K))
    epad = NS * K * nch
    nch_deg = epad // (NC * NS * K)

    ei = jnp.pad(edge_index.astype(jnp.int32), ((0, 0), (0, epad - e)),
                 constant_values=npad - 1)

    ones128 = jnp.ones((K,), jnp.float32)
    zeros1 = jnp.zeros((stripe,), jnp.float32)
    zeros2 = jnp.zeros((K, half), jnp.float32)

    wcat = jnp.concatenate(
        [W_phi.T, (W - W.T - GAMMA * jnp.eye(d, dtype=x.dtype)).T], axis=1)

    deg_call = pl.kernel(
        functools.partial(_deg_body, npad, nch_deg),
        out_type=jax.ShapeDtypeStruct((NC * npad,), jnp.float32),
        mesh=_sc_mesh(),
        scratch_types=[
            pltpu.VMEM((2, K), jnp.int32),
            pltpu.VMEM((2, K), jnp.int32),
            pltpu.VMEM((K,), jnp.float32),
            pltpu.VMEM((stripe,), jnp.float32),
            pltpu.VMEM_SHARED((npad,), jnp.float32),
            pltpu.SemaphoreType.DMA,
            pltpu.SemaphoreType.DMA,
        ],
    )
    degp = deg_call(ei, ones128, zeros1)
    degp0 = degp[:npad].reshape(npad, 1)
    degp1 = degp[npad:].reshape(npad, 1)

    nb = 10
    r = n // nb
    h2, xws0, xws1 = pl.pallas_call(
        _dense_body,
        grid=(nb,),
        in_specs=[
            pl.BlockSpec((r, d), lambda i: (i, 0)),
            pl.BlockSpec((d, 2 * d), lambda i: (0, 0)),
            pl.BlockSpec((r, 1), lambda i: (i, 0)),
            pl.BlockSpec((r, 1), lambda i: (i, 0)),
        ],
        out_specs=[
            pl.BlockSpec((r, d), lambda i: (i, 0)),
            pl.BlockSpec((r, half), lambda i: (i, 0)),
            pl.BlockSpec((r, half), lambda i: (i, 0)),
        ],
        out_shape=[
            jax.ShapeDtypeStruct((n, d), jnp.float32),
            jax.ShapeDtypeStruct((npad, half), jnp.float32),
            jax.ShapeDtypeStruct((npad, half), jnp.float32),
        ],
    )(x, wcat, degp0, degp1)

    gcn_call = pl.kernel(
        functools.partial(_gcn_body, npad, nch),
        out_type=jax.ShapeDtypeStruct((NC, npad, half), jnp.float32),
        mesh=_sc_mesh(),
        scratch_types=[
            [pltpu.VMEM((2, K), jnp.int32) for _ in range(NI)],
            [pltpu.VMEM((K, half), jnp.float32) for _ in range(NG)],
            pltpu.VMEM_SHARED((npad, half), jnp.float32),
            [pltpu.SemaphoreType.DMA for _ in range(NI)],
            [pltpu.SemaphoreType.DMA for _ in range(NG)],
            [pltpu.SemaphoreType.DMA for _ in range(NG)],
        ],
    )
    gcn = gcn_call(xws0, xws1, ei, zeros2)

    out = pl.pallas_call(
        _combine_body,
        grid=(nb,),
        in_specs=[
            pl.BlockSpec((r, d), lambda i: (i, 0)),
            pl.BlockSpec((r, d), lambda i: (i, 0)),
            pl.BlockSpec((2, r, half), lambda i: (0, i, 0)),
            pl.BlockSpec((r, 1), lambda i: (i, 0)),
            pl.BlockSpec((r, 1), lambda i: (i, 0)),
            pl.BlockSpec((1, d), lambda i: (0, 0)),
        ],
        out_specs=pl.BlockSpec((r, d), lambda i: (i, 0)),
        out_shape=jax.ShapeDtypeStruct((n, d), jnp.float32),
    )(x, h2, gcn, degp0, degp1, bias.reshape(1, d))
    return out
